# baseline (speedup 1.0000x reference)
import jax
import jax.numpy as jnp
from jax.experimental import pallas as pl
from jax.experimental.pallas import tpu as pltpu
from jax.experimental.pallas import tpu_sc as plsc

_K = 8192
_H = 256
_TM = 256
_TN = 1024
_BETA = 0.25
_GW = 128


def _argmin_body(z_ref, eT_ref, idx_ref, bd_ref, e2_ref):
    @pl.when(pl.program_id(0) == 0)
    def _():
        eT = eT_ref[...]
        e2_ref[...] = jnp.sum(eT * eT, axis=0, keepdims=True)

    z = z_ref[...]
    z2 = jnp.sum(z * z, axis=1, keepdims=True)
    zm2 = z * (-2.0)
    cols = jax.lax.broadcasted_iota(jnp.int32, (_TM, _TN), 1)
    big = jnp.int32(2 ** 30)

    def step(j, carry):
        best_d, best_i = carry
        eTc = eT_ref[:, pl.ds(j * _TN, _TN)]
        pm2 = jnp.dot(zm2, eTc, preferred_element_type=jnp.float32)
        e2c = e2_ref[:, pl.ds(j * _TN, _TN)]
        dist = (z2 + e2c) + pm2
        cmin = jnp.min(dist, axis=1, keepdims=True)
        lim = jnp.where(dist == cmin, cols, big)
        cidx = jnp.min(lim, axis=1, keepdims=True) + j * _TN
        better = cmin < best_d
        return (jnp.where(better, cmin, best_d),
                jnp.where(better, cidx, best_i))

    init = (jnp.full((_TM, 1), jnp.inf, jnp.float32),
            jnp.zeros((_TM, 1), jnp.int32))
    best_d, best_i = jax.lax.fori_loop(0, _K // _TN, step, init)
    idx_ref[...] = best_i.reshape(1, 1, _TM)
    bd_ref[...] = best_d.reshape(1, 1, _TM)


def _argmin_call(zf, eT):
    n_tiles = zf.shape[0] // _TM
    return pl.pallas_call(
        _argmin_body,
        grid=(n_tiles,),
        in_specs=[
            pl.BlockSpec((_TM, _H), lambda i: (i, 0)),
            pl.BlockSpec((_H, _K), lambda i: (0, 0)),
        ],
        out_specs=[
            pl.BlockSpec((1, 1, _TM), lambda i: (i, 0, 0)),
            pl.BlockSpec((1, 1, _TM), lambda i: (i, 0, 0)),
        ],
        out_shape=[
            jax.ShapeDtypeStruct((n_tiles, 1, _TM), jnp.int32),
            jax.ShapeDtypeStruct((n_tiles, 1, _TM), jnp.float32),
        ],
        scratch_shapes=[pltpu.VMEM((1, _K), jnp.float32)],
    )(zf, eT)


def _sc_gather(emb, idx):
    n = idx.shape[0]
    mesh = plsc.VectorSubcoreMesh(core_axis_name="core",
                                  subcore_axis_name="subcore")
    idx2 = idx.reshape(1, n)

    @pl.kernel(out_type=jax.ShapeDtypeStruct((n, _H), emb.dtype), mesh=mesh)
    def k(emb_hbm, i_hbm, o_hbm):
        def body(i_vmem, o_vmem):
            pltpu.sync_copy(emb_hbm.at[i_vmem.at[0]], o_vmem)

        pltpu.emit_pipeline(
            body,
            grid=(n // _GW,),
            in_specs=[pl.BlockSpec((1, _GW), index_map=lambda i: (0, i))],
            out_specs=[pl.BlockSpec((_GW, _H), index_map=lambda i: (i, 0))],
            core_axis_name=("core", "subcore"),
            dimension_semantics=(pltpu.PARALLEL,),
        )(i_hbm, o_hbm)

    return k(emb, idx2)


def kernel(z_e, embedding):
    zf = z_e.reshape(-1, _H)
    eT = embedding.T
    idx, bd = _argmin_call(zf, eT)
    z_q = _sc_gather(embedding, idx.reshape(-1)).reshape(z_e.shape)
    loss = (1.0 + _BETA) * (jnp.sum(bd) / zf.size)
    return z_q, loss


# speedup vs baseline: 1.5715x; 1.5715x over previous
import jax
import jax.numpy as jnp
from jax.experimental import pallas as pl
from jax.experimental.pallas import tpu as pltpu
from jax.experimental.pallas import tpu_sc as plsc

_K = 8192
_H = 256
_TM = 256
_TN = 1024
_BETA = 0.25
_GW = 128


def _argmin_body(z_ref, eT_ref, idx_ref, bd_ref, e2_ref, cm_ref, ci_ref):
    n_chunks = _K // _TN

    @pl.when(pl.program_id(0) == 0)
    def _():
        eT = eT_ref[...]
        e2_ref[...] = jnp.sum(eT * eT, axis=0, keepdims=True)

    z = z_ref[...]
    z2 = jnp.sum(z * z, axis=1, keepdims=True)
    zm2 = z * (-2.0)
    colf = jax.lax.broadcasted_iota(jnp.int32, (1, _TN), 1).astype(jnp.float32)
    bigf = jnp.float32(1e9)

    for j in range(n_chunks):
        eTc = eT_ref[:, j * _TN:(j + 1) * _TN]
        pm2 = jnp.dot(zm2, eTc, preferred_element_type=jnp.float32)
        e2c = e2_ref[:, j * _TN:(j + 1) * _TN]
        dist = (z2 + e2c) + pm2
        cmin = jnp.min(dist, axis=1, keepdims=True)
        lim = jnp.where(dist == cmin, colf, bigf)
        cidx = jnp.min(lim, axis=1, keepdims=True)
        cm_ref[:, j:j + 1] = cmin
        ci_ref[:, j:j + 1] = cidx

    cm = cm_ref[...]
    best = jnp.min(cm, axis=1, keepdims=True)
    off = (jax.lax.broadcasted_iota(jnp.int32, (1, n_chunks), 1)
           .astype(jnp.float32) * _TN)
    sel = jnp.where(cm == best, ci_ref[...] + off, bigf)
    gidx = jnp.min(sel, axis=1, keepdims=True)
    idx_ref[...] = gidx.astype(jnp.int32).reshape(1, 1, _TM)
    bd_ref[...] = best.reshape(1, 1, _TM)


def _argmin_call(zf, eT):
    n_tiles = zf.shape[0] // _TM
    return pl.pallas_call(
        _argmin_body,
        grid=(n_tiles,),
        in_specs=[
            pl.BlockSpec((_TM, _H), lambda i: (i, 0)),
            pl.BlockSpec((_H, _K), lambda i: (0, 0)),
        ],
        out_specs=[
            pl.BlockSpec((1, 1, _TM), lambda i: (i, 0, 0)),
            pl.BlockSpec((1, 1, _TM), lambda i: (i, 0, 0)),
        ],
        out_shape=[
            jax.ShapeDtypeStruct((n_tiles, 1, _TM), jnp.int32),
            jax.ShapeDtypeStruct((n_tiles, 1, _TM), jnp.float32),
        ],
        scratch_shapes=[
            pltpu.VMEM((1, _K), jnp.float32),
            pltpu.VMEM((_TM, _K // _TN), jnp.float32),
            pltpu.VMEM((_TM, _K // _TN), jnp.float32),
        ],
    )(zf, eT)


def _sc_gather(emb, idx):
    n = idx.shape[0]
    mesh = plsc.VectorSubcoreMesh(core_axis_name="core",
                                  subcore_axis_name="subcore")
    idx2 = idx.reshape(1, n)

    @pl.kernel(out_type=jax.ShapeDtypeStruct((n, _H), emb.dtype), mesh=mesh)
    def k(emb_hbm, i_hbm, o_hbm):
        def body(i_vmem, o_vmem):
            pltpu.sync_copy(emb_hbm.at[i_vmem.at[0]], o_vmem)

        pltpu.emit_pipeline(
            body,
            grid=(n // _GW,),
            in_specs=[pl.BlockSpec((1, _GW), index_map=lambda i: (0, i))],
            out_specs=[pl.BlockSpec((_GW, _H), index_map=lambda i: (i, 0))],
            core_axis_name=("core", "subcore"),
            dimension_semantics=(pltpu.PARALLEL,),
        )(i_hbm, o_hbm)

    return k(emb, idx2)


def kernel(z_e, embedding):
    zf = z_e.reshape(-1, _H)
    eT = embedding.T
    idx, bd = _argmin_call(zf, eT)
    z_q = _sc_gather(embedding, idx.reshape(-1)).reshape(z_e.shape)
    loss = (1.0 + _BETA) * (jnp.sum(bd) / zf.size)
    return z_q, loss
